# baseline (speedup 1.0000x reference)
import functools

import jax
import jax.numpy as jnp
from jax.experimental import pallas as pl
from jax.experimental.pallas import tpu as pltpu

N = 10000
E = 320000
H = 128
D_E = 16

NB = 1000
EB = 2000

_INTERPRET = False


def _leaky(v):
    return jnp.where(v >= 0, v, 0.01 * v)


def _sigmoid(v):
    return 1.0 / (1.0 + jnp.exp(-v))


def _enter_body(x_ref, WeT_ref, be_ref, attr_ref, x1_ref, ar_ref):
    x1 = _leaky(jnp.dot(x_ref[...], WeT_ref[...],
                        preferred_element_type=jnp.float32) + be_ref[...])
    x1_ref[...] = x1
    ar_ref[...] = jnp.sum(x1 * attr_ref[...], axis=1, keepdims=True)


def _node_enter(x, WeT, be, att_r):
    return pl.pallas_call(
        _enter_body,
        grid=(N // NB,),
        in_specs=[
            pl.BlockSpec((NB, H), lambda i: (i, 0)),
            pl.BlockSpec((H, H), lambda i: (0, 0)),
            pl.BlockSpec((1, H), lambda i: (0, 0)),
            pl.BlockSpec((1, H), lambda i: (0, 0)),
        ],
        out_specs=[
            pl.BlockSpec((NB, H), lambda i: (i, 0)),
            pl.BlockSpec((NB, 1), lambda i: (i, 0)),
        ],
        out_shape=[
            jax.ShapeDtypeStruct((N, H), jnp.float32),
            jax.ShapeDtypeStruct((N, 1), jnp.float32),
        ],
        interpret=_INTERPRET,
    )(x, WeT, be, att_r)


def _edge0_body(xj_ref, ea_ref, ard_ref, W1aT_ref, W1bT_ref, attl_ref,
                W2T_ref, msg_ref, ex_ref):
    m = _leaky(jnp.dot(xj_ref[...], W1aT_ref[...],
                       preferred_element_type=jnp.float32)
               + jnp.dot(ea_ref[...], W1bT_ref[...],
                         preferred_element_type=jnp.float32))
    alpha = _leaky(jnp.sum(m * attl_ref[...], axis=1, keepdims=True)
                   + ard_ref[...])
    ex = jnp.exp(alpha)
    msg_ref[...] = jnp.dot(m, W2T_ref[...],
                           preferred_element_type=jnp.float32) * ex
    ex_ref[...] = ex


def _edge0(xj, ea, ardst, W1aT, W1bT, att_l, W2T):
    return pl.pallas_call(
        _edge0_body,
        grid=(E // EB,),
        in_specs=[
            pl.BlockSpec((EB, H), lambda i: (i, 0)),
            pl.BlockSpec((EB, D_E), lambda i: (i, 0)),
            pl.BlockSpec((EB, 1), lambda i: (i, 0)),
            pl.BlockSpec((H, H), lambda i: (0, 0)),
            pl.BlockSpec((D_E, H), lambda i: (0, 0)),
            pl.BlockSpec((1, H), lambda i: (0, 0)),
            pl.BlockSpec((H, H), lambda i: (0, 0)),
        ],
        out_specs=[
            pl.BlockSpec((EB, H), lambda i: (i, 0)),
            pl.BlockSpec((EB, 1), lambda i: (i, 0)),
        ],
        out_shape=[
            jax.ShapeDtypeStruct((E, H), jnp.float32),
            jax.ShapeDtypeStruct((E, 1), jnp.float32),
        ],
        interpret=_INTERPRET,
    )(xj, ea, ardst, W1aT, W1bT, att_l, W2T)


def _gru(h, xprev, wihT_ref, whhT_ref, bih_ref, bhh_ref):
    gi = jnp.dot(h, wihT_ref[...], preferred_element_type=jnp.float32) \
        + bih_ref[...]
    gh = jnp.dot(xprev, whhT_ref[...], preferred_element_type=jnp.float32) \
        + bhh_ref[...]
    r = _sigmoid(gi[:, :H] + gh[:, :H])
    z = _sigmoid(gi[:, H:2 * H] + gh[:, H:2 * H])
    n = jnp.tanh(gi[:, 2 * H:] + r * gh[:, 2 * H:])
    return jnp.maximum((1.0 - z) * n + z * xprev, 0.0)


def _combine_mid_body(acc_ref, den_ref, bias_ref, xprev_ref, wihT_ref,
                      whhT_ref, bih_ref, bhh_ref, WnT_ref, atts_ref,
                      attd_ref, xn_ref, xl_ref, as_ref, ad_ref):
    out = (jnp.sum(acc_ref[...], axis=0)
           / (jnp.sum(den_ref[...], axis=0) + 1e-16)) + bias_ref[...]
    h = jnp.where(out > 0, out, jnp.exp(out) - 1.0)
    xn = _gru(h, xprev_ref[...], wihT_ref, whhT_ref, bih_ref, bhh_ref)
    xn_ref[...] = xn
    xl = jnp.dot(xn, WnT_ref[...], preferred_element_type=jnp.float32)
    xl_ref[...] = xl
    as_ref[...] = jnp.sum(xl * atts_ref[...], axis=1, keepdims=True)
    ad_ref[...] = jnp.sum(xl * attd_ref[...], axis=1, keepdims=True)


def _combine_mid(P, acc, den, bias, xprev, wihT, whhT, bih, bhh,
                 WnT, att_s, att_d):
    return pl.pallas_call(
        _combine_mid_body,
        grid=(N // NB,),
        in_specs=[
            pl.BlockSpec((P, NB, H), lambda i: (0, i, 0)),
            pl.BlockSpec((P, NB, 1), lambda i: (0, i, 0)),
            pl.BlockSpec((1, H), lambda i: (0, 0)),
            pl.BlockSpec((NB, H), lambda i: (i, 0)),
            pl.BlockSpec((H, 3 * H), lambda i: (0, 0)),
            pl.BlockSpec((H, 3 * H), lambda i: (0, 0)),
            pl.BlockSpec((1, 3 * H), lambda i: (0, 0)),
            pl.BlockSpec((1, 3 * H), lambda i: (0, 0)),
            pl.BlockSpec((H, H), lambda i: (0, 0)),
            pl.BlockSpec((1, H), lambda i: (0, 0)),
            pl.BlockSpec((1, H), lambda i: (0, 0)),
        ],
        out_specs=[
            pl.BlockSpec((NB, H), lambda i: (i, 0)),
            pl.BlockSpec((NB, H), lambda i: (i, 0)),
            pl.BlockSpec((NB, 1), lambda i: (i, 0)),
            pl.BlockSpec((NB, 1), lambda i: (i, 0)),
        ],
        out_shape=[
            jax.ShapeDtypeStruct((N, H), jnp.float32),
            jax.ShapeDtypeStruct((N, H), jnp.float32),
            jax.ShapeDtypeStruct((N, 1), jnp.float32),
            jax.ShapeDtypeStruct((N, 1), jnp.float32),
        ],
        interpret=_INTERPRET,
    )(acc, den, bias, xprev, wihT, whhT, bih, bhh, WnT, att_s, att_d)


def _combine_final_body(acc_ref, den_ref, bias_ref, xprev_ref, wihT_ref,
                        whhT_ref, bih_ref, bhh_ref, y_ref):
    out = (jnp.sum(acc_ref[...], axis=0)
           / (jnp.sum(den_ref[...], axis=0) + 1e-16)) + bias_ref[...]
    h = jnp.where(out > 0, out, jnp.exp(out) - 1.0)
    xn = _gru(h, xprev_ref[...], wihT_ref, whhT_ref, bih_ref, bhh_ref)
    mu = jnp.mean(xn, axis=1, keepdims=True)
    var = jnp.mean((xn - mu) ** 2, axis=1, keepdims=True)
    y_ref[...] = (xn - mu) / jnp.sqrt(var + 1e-5)


def _combine_final(P, acc, den, bias, xprev, wihT, whhT, bih, bhh):
    return pl.pallas_call(
        _combine_final_body,
        grid=(N // NB,),
        in_specs=[
            pl.BlockSpec((P, NB, H), lambda i: (0, i, 0)),
            pl.BlockSpec((P, NB, 1), lambda i: (0, i, 0)),
            pl.BlockSpec((1, H), lambda i: (0, 0)),
            pl.BlockSpec((NB, H), lambda i: (i, 0)),
            pl.BlockSpec((H, 3 * H), lambda i: (0, 0)),
            pl.BlockSpec((H, 3 * H), lambda i: (0, 0)),
            pl.BlockSpec((1, 3 * H), lambda i: (0, 0)),
            pl.BlockSpec((1, 3 * H), lambda i: (0, 0)),
        ],
        out_specs=pl.BlockSpec((NB, H), lambda i: (i, 0)),
        out_shape=jax.ShapeDtypeStruct((N, H), jnp.float32),
        interpret=_INTERPRET,
    )(acc, den, bias, xprev, wihT, whhT, bih, bhh)


def kernel(x, edge_index, edge_attr, params):
    p = params
    src = edge_index[0]
    dst = edge_index[1]

    x1, ar = _node_enter(x, p["lin_enter_W"].T, p["lin_enter_b"][None],
                         p["att_r"])

    xj = x1[src]
    ardst = ar[dst]
    msgw, ex = _edge0(xj, edge_attr, ardst,
                      p["lin1_W"][:, :H].T, p["lin1_W"][:, H:].T,
                      p["att_l"], p["lin2_W"].T)
    acc = jnp.zeros((N, H), jnp.float32).at[dst].add(msgw)
    den = jnp.zeros((N, 1), jnp.float32).at[dst].add(ex)
    xn, xl, asrc, adst = _combine_mid(
        1, acc[None], den[None], p["conv_bias"][None], x1,
        p["gru0_w_ih"].T, p["gru0_w_hh"].T,
        p["gru0_b_ih"][None], p["gru0_b_hh"][None],
        p["gat1_W"].T, p["gat1_att_src"], p["gat1_att_dst"])

    for l in (1, 2):
        exl = jnp.exp(_leaky(asrc[src, 0] + adst[dst, 0]))
        accl = jnp.zeros((N, H), jnp.float32).at[dst].add(xl[src] * exl[:, None])
        denl = jnp.zeros((N, 1), jnp.float32).at[dst].add(exl[:, None])
        if l == 1:
            xn, xl, asrc, adst = _combine_mid(
                1, accl[None], denl[None], p["gat1_bias"][None], xn,
                p["gru1_w_ih"].T, p["gru1_w_hh"].T,
                p["gru1_b_ih"][None], p["gru1_b_hh"][None],
                p["gat2_W"].T, p["gat2_att_src"], p["gat2_att_dst"])
        else:
            y = _combine_final(
                1, accl[None], denl[None], p["gat2_bias"][None], xn,
                p["gru2_w_ih"].T, p["gru2_w_hh"].T,
                p["gru2_b_ih"][None], p["gru2_b_hh"][None])
    return y


# speedup vs baseline: 12.7814x; 12.7814x over previous
import functools

import jax
import jax.numpy as jnp
from jax import lax
from jax.experimental import pallas as pl
from jax.experimental.pallas import tpu as pltpu
from jax.experimental.pallas import tpu_sc as plsc

N = 10000
E = 320000
H = 128
D_E = 16

NB = 1000
EB = 2000

NSC = 2
NTL = 16
NW = NSC * NTL
EPW = E // NW
CH = 80
NCH = EPW // CH
NPAD = 10240
RPT = NPAD // NTL

_SC_MESH = plsc.VectorSubcoreMesh(core_axis_name="c", subcore_axis_name="s",
                                  num_cores=NSC, num_subcores=NTL)

_INTERPRET = False


def _leaky(v):
    return jnp.where(v >= 0, v, 0.01 * v)


def _sigmoid(v):
    return 1.0 / (1.0 + jnp.exp(-v))


def _enter_body(x_ref, WeT_ref, be_ref, attr_ref, x1_ref, ar_ref):
    x1 = _leaky(jnp.dot(x_ref[...], WeT_ref[...],
                        preferred_element_type=jnp.float32) + be_ref[...])
    x1_ref[...] = x1
    ar_ref[...] = jnp.sum(x1 * attr_ref[...], axis=1, keepdims=True)


def _node_enter(x, WeT, be, att_r):
    return pl.pallas_call(
        _enter_body,
        grid=(N // NB,),
        in_specs=[
            pl.BlockSpec((NB, H), lambda i: (i, 0)),
            pl.BlockSpec((H, H), lambda i: (0, 0)),
            pl.BlockSpec((1, H), lambda i: (0, 0)),
            pl.BlockSpec((1, H), lambda i: (0, 0)),
        ],
        out_specs=[
            pl.BlockSpec((NB, H), lambda i: (i, 0)),
            pl.BlockSpec((NB, 1), lambda i: (i, 0)),
        ],
        out_shape=[
            jax.ShapeDtypeStruct((N, H), jnp.float32),
            jax.ShapeDtypeStruct((N, 1), jnp.float32),
        ],
        interpret=_INTERPRET,
    )(x, WeT, be, att_r)


def _edge0_body(xj_ref, ea_ref, ard_ref, W1aT_ref, W1bT_ref, attl_ref,
                W2T_ref, msg_ref, ex_ref):
    m = _leaky(jnp.dot(xj_ref[...], W1aT_ref[...],
                       preferred_element_type=jnp.float32)
               + jnp.dot(ea_ref[...], W1bT_ref[...],
                         preferred_element_type=jnp.float32))
    alpha = _leaky(jnp.sum(m * attl_ref[...], axis=1, keepdims=True)
                   + ard_ref[...])
    ex = jnp.exp(alpha)
    msg_ref[...] = jnp.dot(m, W2T_ref[...],
                           preferred_element_type=jnp.float32) * ex
    ex_ref[...] = ex


def _edge0(xj, ea, ardst, W1aT, W1bT, att_l, W2T):
    return pl.pallas_call(
        _edge0_body,
        grid=(E // EB,),
        in_specs=[
            pl.BlockSpec((EB, H), lambda i: (i, 0)),
            pl.BlockSpec((EB, D_E), lambda i: (i, 0)),
            pl.BlockSpec((EB, 1), lambda i: (i, 0)),
            pl.BlockSpec((H, H), lambda i: (0, 0)),
            pl.BlockSpec((D_E, H), lambda i: (0, 0)),
            pl.BlockSpec((1, H), lambda i: (0, 0)),
            pl.BlockSpec((H, H), lambda i: (0, 0)),
        ],
        out_specs=[
            pl.BlockSpec((EB, H), lambda i: (i, 0)),
            pl.BlockSpec((EB, 1), lambda i: (i, 0)),
        ],
        out_shape=[
            jax.ShapeDtypeStruct((E, H), jnp.float32),
            jax.ShapeDtypeStruct((E, 1), jnp.float32),
        ],
        interpret=_INTERPRET,
    )(xj, ea, ardst, W1aT, W1bT, att_l, W2T)


def _gru(h, xprev, wihT_ref, whhT_ref, bih_ref, bhh_ref):
    gi = jnp.dot(h, wihT_ref[...], preferred_element_type=jnp.float32) \
        + bih_ref[...]
    gh = jnp.dot(xprev, whhT_ref[...], preferred_element_type=jnp.float32) \
        + bhh_ref[...]
    r = _sigmoid(gi[:, :H] + gh[:, :H])
    z = _sigmoid(gi[:, H:2 * H] + gh[:, H:2 * H])
    n = jnp.tanh(gi[:, 2 * H:] + r * gh[:, 2 * H:])
    return jnp.maximum((1.0 - z) * n + z * xprev, 0.0)


def _combine_mid_body(acc_ref, den_ref, bias_ref, xprev_ref, wihT_ref,
                      whhT_ref, bih_ref, bhh_ref, WnT_ref, atts_ref,
                      attd_ref, xn_ref, xl_ref, as_ref, ad_ref):
    out = (jnp.sum(acc_ref[...], axis=0)
           / (jnp.sum(den_ref[...], axis=0) + 1e-16)) + bias_ref[...]
    h = jnp.where(out > 0, out, jnp.exp(out) - 1.0)
    xn = _gru(h, xprev_ref[...], wihT_ref, whhT_ref, bih_ref, bhh_ref)
    xn_ref[...] = xn
    xl = jnp.dot(xn, WnT_ref[...], preferred_element_type=jnp.float32)
    xl_ref[...] = xl
    as_ref[...] = jnp.sum(xl * atts_ref[...], axis=1, keepdims=True)
    ad_ref[...] = jnp.sum(xl * attd_ref[...], axis=1, keepdims=True)


def _combine_mid(P, acc, den, bias, xprev, wihT, whhT, bih, bhh,
                 WnT, att_s, att_d):
    return pl.pallas_call(
        _combine_mid_body,
        grid=(N // NB,),
        in_specs=[
            pl.BlockSpec((P, NB, H), lambda i: (0, i, 0)),
            pl.BlockSpec((P, NB, 1), lambda i: (0, i, 0)),
            pl.BlockSpec((1, H), lambda i: (0, 0)),
            pl.BlockSpec((NB, H), lambda i: (i, 0)),
            pl.BlockSpec((H, 3 * H), lambda i: (0, 0)),
            pl.BlockSpec((H, 3 * H), lambda i: (0, 0)),
            pl.BlockSpec((1, 3 * H), lambda i: (0, 0)),
            pl.BlockSpec((1, 3 * H), lambda i: (0, 0)),
            pl.BlockSpec((H, H), lambda i: (0, 0)),
            pl.BlockSpec((1, H), lambda i: (0, 0)),
            pl.BlockSpec((1, H), lambda i: (0, 0)),
        ],
        out_specs=[
            pl.BlockSpec((NB, H), lambda i: (i, 0)),
            pl.BlockSpec((NB, H), lambda i: (i, 0)),
            pl.BlockSpec((NB, 1), lambda i: (i, 0)),
            pl.BlockSpec((NB, 1), lambda i: (i, 0)),
        ],
        out_shape=[
            jax.ShapeDtypeStruct((N, H), jnp.float32),
            jax.ShapeDtypeStruct((N, H), jnp.float32),
            jax.ShapeDtypeStruct((N, 1), jnp.float32),
            jax.ShapeDtypeStruct((N, 1), jnp.float32),
        ],
        interpret=_INTERPRET,
    )(acc, den, bias, xprev, wihT, whhT, bih, bhh, WnT, att_s, att_d)


def _combine_final_body(acc_ref, den_ref, bias_ref, xprev_ref, wihT_ref,
                        whhT_ref, bih_ref, bhh_ref, y_ref):
    out = (jnp.sum(acc_ref[...], axis=0)
           / (jnp.sum(den_ref[...], axis=0) + 1e-16)) + bias_ref[...]
    h = jnp.where(out > 0, out, jnp.exp(out) - 1.0)
    xn = _gru(h, xprev_ref[...], wihT_ref, whhT_ref, bih_ref, bhh_ref)
    mu = jnp.mean(xn, axis=1, keepdims=True)
    var = jnp.mean((xn - mu) ** 2, axis=1, keepdims=True)
    y_ref[...] = (xn - mu) / jnp.sqrt(var + 1e-5)


def _combine_final(P, acc, den, bias, xprev, wihT, whhT, bih, bhh):
    return pl.pallas_call(
        _combine_final_body,
        grid=(N // NB,),
        in_specs=[
            pl.BlockSpec((P, NB, H), lambda i: (0, i, 0)),
            pl.BlockSpec((P, NB, 1), lambda i: (0, i, 0)),
            pl.BlockSpec((1, H), lambda i: (0, 0)),
            pl.BlockSpec((NB, H), lambda i: (i, 0)),
            pl.BlockSpec((H, 3 * H), lambda i: (0, 0)),
            pl.BlockSpec((H, 3 * H), lambda i: (0, 0)),
            pl.BlockSpec((1, 3 * H), lambda i: (0, 0)),
            pl.BlockSpec((1, 3 * H), lambda i: (0, 0)),
        ],
        out_specs=pl.BlockSpec((NB, H), lambda i: (i, 0)),
        out_shape=jax.ShapeDtypeStruct((N, H), jnp.float32),
        interpret=_INTERPRET,
    )(acc, den, bias, xprev, wihT, whhT, bih, bhh)


def _zero_vmem_rows(zrow_v):
    for k in range(zrow_v.shape[0]):
        for cc in range(H // 16):
            zrow_v[k, pl.ds(cc * 16, 16)] = jnp.zeros((16,), jnp.float32)


def _zero_vmem_vec(zden_v):
    def zden(k, carry):
        zden_v[pl.ds(k * 16, 16)] = jnp.zeros((16,), jnp.float32)
        return carry
    lax.fori_loop(0, zden_v.shape[0] // 16, zden, 0)


def _zero_shared(s, zrow_v, zden_v, acc_sh, den_sh):
    _zero_vmem_rows(zrow_v)
    _zero_vmem_vec(zden_v)
    zr = zrow_v.shape[0]

    def zcopy(r, carry):
        pltpu.sync_copy(zrow_v, acc_sh.at[pl.ds(s * RPT + r * zr, zr)])
        return carry

    lax.fori_loop(0, RPT // zr, zcopy, 0)
    pltpu.sync_copy(zden_v, den_sh.at[pl.ds(s * RPT, RPT)])


def _sc_gather0_body(x1_hbm, ar_hbm, src_hbm, dst_hbm, xj_hbm, ard_hbm,
                     idx_v, didx_v, rows_v, ar_tab, ard_v, sem):
    c = lax.axis_index("c")
    s = lax.axis_index("s")
    base = (c * NTL + s) * EPW
    pltpu.sync_copy(ar_hbm, ar_tab)

    def chunk(k, carry):
        off = base + k * CH
        pltpu.sync_copy(src_hbm.at[pl.ds(off, CH)], idx_v)
        pltpu.sync_copy(dst_hbm.at[pl.ds(off, CH)], didx_v)
        pltpu.async_copy(x1_hbm.at[idx_v], rows_v, sem).wait()
        pltpu.sync_copy(rows_v, xj_hbm.at[pl.ds(off, CH)])
        for j in range(CH // 16):
            ids = didx_v[pl.ds(j * 16, 16)]
            ard_v[pl.ds(j * 16, 16)] = plsc.load_gather(ar_tab, [ids])
        pltpu.sync_copy(ard_v, ard_hbm.at[pl.ds(off, CH)])
        return carry

    lax.fori_loop(0, NCH, chunk, 0)


def _sc_gather0(x1, ar, src, dst):
    return pl.kernel(
        _sc_gather0_body,
        out_type=[jax.ShapeDtypeStruct((E, H), jnp.float32),
                  jax.ShapeDtypeStruct((E,), jnp.float32)],
        mesh=_SC_MESH,
        compiler_params=pltpu.CompilerParams(needs_layout_passes=False),
        scratch_types=[pltpu.VMEM((CH,), jnp.int32),
                       pltpu.VMEM((CH,), jnp.int32),
                       pltpu.VMEM((CH, H), jnp.float32),
                       pltpu.VMEM((N,), jnp.float32),
                       pltpu.VMEM((CH,), jnp.float32),
                       pltpu.SemaphoreType.DMA],
    )(x1, ar, src, dst)


def _sc_scatter0_body(msg_hbm, ex_hbm, dst_hbm, acc_out, den_out,
                      rows_v, ex_v, didx_v, zrow_v, zden_v,
                      acc_sh, den_sh, sem):
    c = lax.axis_index("c")
    s = lax.axis_index("s")
    _zero_shared(s, zrow_v, zden_v, acc_sh, den_sh)
    plsc.subcore_barrier()

    base = (c * NTL + s) * EPW

    def chunk(k, carry):
        off = base + k * CH
        pltpu.sync_copy(dst_hbm.at[pl.ds(off, CH)], didx_v)
        pltpu.sync_copy(msg_hbm.at[pl.ds(off, CH)], rows_v)
        pltpu.sync_copy(ex_hbm.at[pl.ds(off, CH)], ex_v)
        pltpu.sync_copy(rows_v, acc_sh.at[didx_v], add=True)
        pltpu.sync_copy(ex_v, den_sh.at[didx_v], add=True)
        return carry

    lax.fori_loop(0, NCH, chunk, 0)
    plsc.subcore_barrier()
    pltpu.sync_copy(acc_sh.at[pl.ds(s * RPT, RPT)],
                    acc_out.at[c, pl.ds(s * RPT, RPT)])
    pltpu.sync_copy(den_sh.at[pl.ds(s * RPT, RPT)],
                    den_out.at[c, pl.ds(s * RPT, RPT)])


def _sc_scatter0(msgw, ex, dst):
    return pl.kernel(
        _sc_scatter0_body,
        out_type=[jax.ShapeDtypeStruct((NSC, NPAD, H), jnp.float32),
                  jax.ShapeDtypeStruct((NSC, NPAD), jnp.float32)],
        mesh=_SC_MESH,
        compiler_params=pltpu.CompilerParams(needs_layout_passes=False),
        scratch_types=[pltpu.VMEM((CH, H), jnp.float32),
                       pltpu.VMEM((CH,), jnp.float32),
                       pltpu.VMEM((CH,), jnp.int32),
                       pltpu.VMEM((16, H), jnp.float32),
                       pltpu.VMEM((RPT,), jnp.float32),
                       pltpu.VMEM_SHARED((NPAD, H), jnp.float32),
                       pltpu.VMEM_SHARED((NPAD,), jnp.float32),
                       pltpu.SemaphoreType.DMA],
    )(msgw, ex, dst)


def _sc_gat_body(xl_hbm, as_hbm, ad_hbm, src_hbm, dst_hbm, acc_out, den_out,
                 sidx_v, didx_v, rows_v, ex_v, as_tab, ad_tab,
                 zrow_v, zden_v, acc_sh, den_sh, sem):
    c = lax.axis_index("c")
    s = lax.axis_index("s")
    _zero_shared(s, zrow_v, zden_v, acc_sh, den_sh)
    pltpu.sync_copy(as_hbm, as_tab)
    pltpu.sync_copy(ad_hbm, ad_tab)
    plsc.subcore_barrier()

    base = (c * NTL + s) * EPW

    def chunk(k, carry):
        off = base + k * CH
        pltpu.sync_copy(src_hbm.at[pl.ds(off, CH)], sidx_v)
        pltpu.sync_copy(dst_hbm.at[pl.ds(off, CH)], didx_v)
        pltpu.async_copy(xl_hbm.at[sidx_v], rows_v, sem).wait()
        for j in range(CH // 16):
            sid = sidx_v[pl.ds(j * 16, 16)]
            did = didx_v[pl.ds(j * 16, 16)]
            a = plsc.load_gather(as_tab, [sid]) \
                + plsc.load_gather(ad_tab, [did])
            a = jnp.where(a >= 0, a, 0.01 * a)
            ex16 = jnp.exp(a)
            ex_v[pl.ds(j * 16, 16)] = ex16
            for t in range(16):
                e = j * 16 + t
                w = ex16[t]
                for cc in range(H // 16):
                    rows_v[e, pl.ds(cc * 16, 16)] = \
                        rows_v[e, pl.ds(cc * 16, 16)] * w
        pltpu.sync_copy(rows_v, acc_sh.at[didx_v], add=True)
        pltpu.sync_copy(ex_v, den_sh.at[didx_v], add=True)
        return carry

    lax.fori_loop(0, NCH, chunk, 0)
    plsc.subcore_barrier()
    pltpu.sync_copy(acc_sh.at[pl.ds(s * RPT, RPT)],
                    acc_out.at[c, pl.ds(s * RPT, RPT)])
    pltpu.sync_copy(den_sh.at[pl.ds(s * RPT, RPT)],
                    den_out.at[c, pl.ds(s * RPT, RPT)])


def _sc_gat(xl, asrc, adst, src, dst):
    return pl.kernel(
        _sc_gat_body,
        out_type=[jax.ShapeDtypeStruct((NSC, NPAD, H), jnp.float32),
                  jax.ShapeDtypeStruct((NSC, NPAD), jnp.float32)],
        mesh=_SC_MESH,
        compiler_params=pltpu.CompilerParams(needs_layout_passes=False),
        scratch_types=[pltpu.VMEM((CH,), jnp.int32),
                       pltpu.VMEM((CH,), jnp.int32),
                       pltpu.VMEM((CH, H), jnp.float32),
                       pltpu.VMEM((CH,), jnp.float32),
                       pltpu.VMEM((N,), jnp.float32),
                       pltpu.VMEM((N,), jnp.float32),
                       pltpu.VMEM((16, H), jnp.float32),
                       pltpu.VMEM((RPT,), jnp.float32),
                       pltpu.VMEM_SHARED((NPAD, H), jnp.float32),
                       pltpu.VMEM_SHARED((NPAD,), jnp.float32),
                       pltpu.SemaphoreType.DMA],
    )(xl, asrc, adst, src, dst)


def kernel(x, edge_index, edge_attr, params):
    p = params
    src = edge_index[0]
    dst = edge_index[1]

    x1, ar = _node_enter(x, p["lin_enter_W"].T, p["lin_enter_b"][None],
                         p["att_r"])

    xj, ardst = _sc_gather0(x1, ar.reshape(N), src, dst)
    msgw, ex = _edge0(xj, edge_attr, ardst.reshape(E, 1),
                      p["lin1_W"][:, :H].T, p["lin1_W"][:, H:].T,
                      p["att_l"], p["lin2_W"].T)
    acc, den = _sc_scatter0(msgw, ex.reshape(E), dst)
    xn, xl, asrc, adst = _combine_mid(
        NSC, acc, den.reshape(NSC, NPAD, 1), p["conv_bias"][None], x1,
        p["gru0_w_ih"].T, p["gru0_w_hh"].T,
        p["gru0_b_ih"][None], p["gru0_b_hh"][None],
        p["gat1_W"].T, p["gat1_att_src"], p["gat1_att_dst"])

    for l in (1, 2):
        accl, denl = _sc_gat(xl, asrc.reshape(N), adst.reshape(N), src, dst)
        denl = denl.reshape(NSC, NPAD, 1)
        if l == 1:
            xn, xl, asrc, adst = _combine_mid(
                NSC, accl, denl, p["gat1_bias"][None], xn,
                p["gru1_w_ih"].T, p["gru1_w_hh"].T,
                p["gru1_b_ih"][None], p["gru1_b_hh"][None],
                p["gat2_W"].T, p["gat2_att_src"], p["gat2_att_dst"])
        else:
            y = _combine_final(
                NSC, accl, denl, p["gat2_bias"][None], xn,
                p["gru2_w_ih"].T, p["gru2_w_hh"].T,
                p["gru2_b_ih"][None], p["gru2_b_hh"][None])
    return y


# speedup vs baseline: 19.0756x; 1.4925x over previous
import functools

import jax
import jax.numpy as jnp
from jax import lax
from jax.experimental import pallas as pl
from jax.experimental.pallas import tpu as pltpu
from jax.experimental.pallas import tpu_sc as plsc

N = 10000
E = 320000
H = 128
D_E = 16

NB = 1000
EB = 2000

NSC = 2
NTL = 16
NW = NSC * NTL
EPW = E // NW
CH = 80
NCH = EPW // CH
NPAD = 10240
RPT = NPAD // NTL

@functools.lru_cache(maxsize=1)
def _sc_mesh():
    return plsc.VectorSubcoreMesh(core_axis_name="c", subcore_axis_name="s",
                                  num_cores=NSC, num_subcores=NTL)

_INTERPRET = False


def _leaky(v):
    return jnp.where(v >= 0, v, 0.01 * v)


def _sigmoid(v):
    return 1.0 / (1.0 + jnp.exp(-v))


def _enter_body(x_ref, WeT_ref, be_ref, attr_ref, W1aT_ref,
                x1_ref, ar_ref, xw_ref):
    x1 = _leaky(jnp.dot(x_ref[...], WeT_ref[...],
                        preferred_element_type=jnp.float32) + be_ref[...])
    x1_ref[...] = x1
    ar_ref[...] = jnp.sum(x1 * attr_ref[...], axis=1, keepdims=True)
    xw_ref[...] = jnp.dot(x1, W1aT_ref[...],
                          preferred_element_type=jnp.float32)


def _node_enter(x, WeT, be, att_r, W1aT):
    return pl.pallas_call(
        _enter_body,
        grid=(N // NB,),
        in_specs=[
            pl.BlockSpec((NB, H), lambda i: (i, 0)),
            pl.BlockSpec((H, H), lambda i: (0, 0)),
            pl.BlockSpec((1, H), lambda i: (0, 0)),
            pl.BlockSpec((1, H), lambda i: (0, 0)),
            pl.BlockSpec((H, H), lambda i: (0, 0)),
        ],
        out_specs=[
            pl.BlockSpec((NB, H), lambda i: (i, 0)),
            pl.BlockSpec((NB, 1), lambda i: (i, 0)),
            pl.BlockSpec((NB, H), lambda i: (i, 0)),
        ],
        out_shape=[
            jax.ShapeDtypeStruct((N, H), jnp.float32),
            jax.ShapeDtypeStruct((N, 1), jnp.float32),
            jax.ShapeDtypeStruct((N, H), jnp.float32),
        ],
        interpret=_INTERPRET,
    )(x, WeT, be, att_r, W1aT)


def _edge0_body(xjw_ref, ea_ref, ard_ref, W1bT_ref, attl_ref,
                msg_ref, ex_ref):
    m = _leaky(xjw_ref[...]
               + jnp.dot(ea_ref[...], W1bT_ref[...],
                         preferred_element_type=jnp.float32))
    alpha = _leaky(jnp.sum(m * attl_ref[...], axis=1, keepdims=True)
                   + ard_ref[...])
    ex = jnp.exp(alpha)
    msg_ref[...] = m * ex
    ex_ref[...] = ex


def _edge0(xjw, ea, ardst, W1bT, att_l):
    return pl.pallas_call(
        _edge0_body,
        grid=(E // EB,),
        in_specs=[
            pl.BlockSpec((EB, H), lambda i: (i, 0)),
            pl.BlockSpec((EB, D_E), lambda i: (i, 0)),
            pl.BlockSpec((EB, 1), lambda i: (i, 0)),
            pl.BlockSpec((D_E, H), lambda i: (0, 0)),
            pl.BlockSpec((1, H), lambda i: (0, 0)),
        ],
        out_specs=[
            pl.BlockSpec((EB, H), lambda i: (i, 0)),
            pl.BlockSpec((EB, 1), lambda i: (i, 0)),
        ],
        out_shape=[
            jax.ShapeDtypeStruct((E, H), jnp.float32),
            jax.ShapeDtypeStruct((E, 1), jnp.float32),
        ],
        interpret=_INTERPRET,
    )(xjw, ea, ardst, W1bT, att_l)


def _gru(h, xprev, wihT_ref, whhT_ref, bih_ref, bhh_ref):
    gi = jnp.dot(h, wihT_ref[...], preferred_element_type=jnp.float32) \
        + bih_ref[...]
    gh = jnp.dot(xprev, whhT_ref[...], preferred_element_type=jnp.float32) \
        + bhh_ref[...]
    r = _sigmoid(gi[:, :H] + gh[:, :H])
    z = _sigmoid(gi[:, H:2 * H] + gh[:, H:2 * H])
    n = jnp.tanh(gi[:, 2 * H:] + r * gh[:, 2 * H:])
    return jnp.maximum((1.0 - z) * n + z * xprev, 0.0)


def _combine_mid(P, acc, den, bias, xprev, wihT, whhT, bih, bhh,
                 WnT, att_s, att_d, preW=None):
    has_pre = preW is not None

    def body(*refs):
        if has_pre:
            (acc_ref, den_ref, preW_ref, bias_ref, xprev_ref, wihT_ref,
             whhT_ref, bih_ref, bhh_ref, WnT_ref, atts_ref, attd_ref,
             xn_ref, xl_ref, as_ref, ad_ref) = refs
        else:
            (acc_ref, den_ref, bias_ref, xprev_ref, wihT_ref,
             whhT_ref, bih_ref, bhh_ref, WnT_ref, atts_ref, attd_ref,
             xn_ref, xl_ref, as_ref, ad_ref) = refs
        out = (jnp.sum(acc_ref[...], axis=0)
               / (jnp.sum(den_ref[...], axis=0) + 1e-16))
        if has_pre:
            out = jnp.dot(out, preW_ref[...],
                          preferred_element_type=jnp.float32)
        out = out + bias_ref[...]
        h = jnp.where(out > 0, out, jnp.exp(out) - 1.0)
        xn = _gru(h, xprev_ref[...], wihT_ref, whhT_ref, bih_ref, bhh_ref)
        xn_ref[...] = xn
        xl = jnp.dot(xn, WnT_ref[...], preferred_element_type=jnp.float32)
        xl_ref[...] = xl
        as_ref[...] = jnp.sum(xl * atts_ref[...], axis=1, keepdims=True)
        ad_ref[...] = jnp.sum(xl * attd_ref[...], axis=1, keepdims=True)

    in_specs = [
        pl.BlockSpec((P, NB, H), lambda i: (0, i, 0)),
        pl.BlockSpec((P, NB, 1), lambda i: (0, i, 0)),
    ]
    args = [acc, den]
    if has_pre:
        in_specs.append(pl.BlockSpec((H, H), lambda i: (0, 0)))
        args.append(preW)
    in_specs += [
        pl.BlockSpec((1, H), lambda i: (0, 0)),
        pl.BlockSpec((NB, H), lambda i: (i, 0)),
        pl.BlockSpec((H, 3 * H), lambda i: (0, 0)),
        pl.BlockSpec((H, 3 * H), lambda i: (0, 0)),
        pl.BlockSpec((1, 3 * H), lambda i: (0, 0)),
        pl.BlockSpec((1, 3 * H), lambda i: (0, 0)),
        pl.BlockSpec((H, H), lambda i: (0, 0)),
        pl.BlockSpec((1, H), lambda i: (0, 0)),
        pl.BlockSpec((1, H), lambda i: (0, 0)),
    ]
    args += [bias, xprev, wihT, whhT, bih, bhh, WnT, att_s, att_d]
    return pl.pallas_call(
        body,
        grid=(N // NB,),
        in_specs=in_specs,
        out_specs=[
            pl.BlockSpec((NB, H), lambda i: (i, 0)),
            pl.BlockSpec((NB, H), lambda i: (i, 0)),
            pl.BlockSpec((NB, 1), lambda i: (i, 0)),
            pl.BlockSpec((NB, 1), lambda i: (i, 0)),
        ],
        out_shape=[
            jax.ShapeDtypeStruct((N, H), jnp.float32),
            jax.ShapeDtypeStruct((N, H), jnp.float32),
            jax.ShapeDtypeStruct((N, 1), jnp.float32),
            jax.ShapeDtypeStruct((N, 1), jnp.float32),
        ],
        interpret=_INTERPRET,
    )(*args)


def _combine_final_body(acc_ref, den_ref, bias_ref, xprev_ref, wihT_ref,
                        whhT_ref, bih_ref, bhh_ref, y_ref):
    out = (jnp.sum(acc_ref[...], axis=0)
           / (jnp.sum(den_ref[...], axis=0) + 1e-16)) + bias_ref[...]
    h = jnp.where(out > 0, out, jnp.exp(out) - 1.0)
    xn = _gru(h, xprev_ref[...], wihT_ref, whhT_ref, bih_ref, bhh_ref)
    mu = jnp.mean(xn, axis=1, keepdims=True)
    var = jnp.mean((xn - mu) ** 2, axis=1, keepdims=True)
    y_ref[...] = (xn - mu) / jnp.sqrt(var + 1e-5)


def _combine_final(P, acc, den, bias, xprev, wihT, whhT, bih, bhh):
    return pl.pallas_call(
        _combine_final_body,
        grid=(N // NB,),
        in_specs=[
            pl.BlockSpec((P, NB, H), lambda i: (0, i, 0)),
            pl.BlockSpec((P, NB, 1), lambda i: (0, i, 0)),
            pl.BlockSpec((1, H), lambda i: (0, 0)),
            pl.BlockSpec((NB, H), lambda i: (i, 0)),
            pl.BlockSpec((H, 3 * H), lambda i: (0, 0)),
            pl.BlockSpec((H, 3 * H), lambda i: (0, 0)),
            pl.BlockSpec((1, 3 * H), lambda i: (0, 0)),
            pl.BlockSpec((1, 3 * H), lambda i: (0, 0)),
        ],
        out_specs=pl.BlockSpec((NB, H), lambda i: (i, 0)),
        out_shape=jax.ShapeDtypeStruct((N, H), jnp.float32),
        interpret=_INTERPRET,
    )(acc, den, bias, xprev, wihT, whhT, bih, bhh)


def _zero_vmem_rows(zrow_v):
    for k in range(zrow_v.shape[0]):
        for cc in range(H // 16):
            zrow_v[k, pl.ds(cc * 16, 16)] = jnp.zeros((16,), jnp.float32)


def _zero_vmem_vec(zden_v):
    def zden(k, carry):
        zden_v[pl.ds(k * 16, 16)] = jnp.zeros((16,), jnp.float32)
        return carry
    lax.fori_loop(0, zden_v.shape[0] // 16, zden, 0)


def _zero_shared(s, zrow_v, zden_v, acc_sh, den_sh):
    _zero_vmem_rows(zrow_v)
    _zero_vmem_vec(zden_v)
    zr = zrow_v.shape[0]

    def zcopy(r, carry):
        pltpu.sync_copy(zrow_v, acc_sh.at[pl.ds(s * RPT + r * zr, zr)])
        return carry

    lax.fori_loop(0, RPT // zr, zcopy, 0)
    pltpu.sync_copy(zden_v, den_sh.at[pl.ds(s * RPT, RPT)])


def _idx_load(src_hbm, dst_hbm, sidx_v, didx_v, off, b, isem):
    pltpu.async_copy(src_hbm.at[pl.ds(off, CH)], sidx_v.at[b], isem)
    pltpu.async_copy(dst_hbm.at[pl.ds(off, CH)], didx_v.at[b], isem)


def _idx_load_sync(src_hbm, dst_hbm, sidx_v, didx_v, off, b):
    pltpu.sync_copy(src_hbm.at[pl.ds(off, CH)], sidx_v.at[b])
    pltpu.sync_copy(dst_hbm.at[pl.ds(off, CH)], didx_v.at[b])


def _idx_wait(src_hbm, dst_hbm, sidx_v, didx_v, b, isem):
    pltpu.make_async_copy(src_hbm.at[pl.ds(0, CH)], sidx_v.at[b], isem).wait()
    pltpu.make_async_copy(dst_hbm.at[pl.ds(0, CH)], didx_v.at[b], isem).wait()


def _sc_gather0_body(x1_hbm, ar_hbm, src_hbm, dst_hbm, xj_hbm, ard_hbm,
                     sidx_v, didx_v, rows_v, ar_tab, ard_v, gsem, isem):
    c = lax.axis_index("c")
    s = lax.axis_index("s")
    base = (c * NTL + s) * EPW
    pltpu.sync_copy(ar_hbm, ar_tab)

    def gather(b):
        pltpu.async_copy(x1_hbm.at[sidx_v.at[b]], rows_v.at[b], gsem)

    def gwait(b):
        pltpu.make_async_copy(x1_hbm.at[sidx_v.at[b]], rows_v.at[b],
                              gsem).wait()

    _idx_load_sync(src_hbm, dst_hbm, sidx_v, didx_v, base, 0)
    _idx_load_sync(src_hbm, dst_hbm, sidx_v, didx_v, base + CH, 1)
    gather(0)

    def body(g, carry):
        for b in range(2):
            k = g * 2 + b

            @pl.when(k < NCH)
            def _():
                bo = (b + 1) % 2
                gwait(b)

                @pl.when(k + 1 < NCH)
                def _():
                    @pl.when(k + 1 >= 2)
                    def _():
                        _idx_wait(src_hbm, dst_hbm, sidx_v, didx_v, bo, isem)
                    gather(bo)

                for j in range(CH // 16):
                    ids = didx_v[b, pl.ds(j * 16, 16)]
                    ard_v[b, pl.ds(j * 16, 16)] = \
                        plsc.load_gather(ar_tab, [ids])
                off = base + k * CH
                pltpu.sync_copy(rows_v.at[b], xj_hbm.at[pl.ds(off, CH)])
                pltpu.sync_copy(ard_v.at[b], ard_hbm.at[pl.ds(off, CH)])

                @pl.when(k + 2 < NCH)
                def _():
                    _idx_load(src_hbm, dst_hbm, sidx_v, didx_v,
                              base + (k + 2) * CH, b, isem)
        return carry

    lax.fori_loop(0, (NCH + 1) // 2, body, 0)


def _sc_gather0(x1, ar, src, dst):
    return pl.kernel(
        _sc_gather0_body,
        out_type=[jax.ShapeDtypeStruct((E, H), jnp.float32),
                  jax.ShapeDtypeStruct((E,), jnp.float32)],
        mesh=_sc_mesh(),
        compiler_params=pltpu.CompilerParams(needs_layout_passes=False),
        scratch_types=[pltpu.VMEM((2, CH), jnp.int32),
                       pltpu.VMEM((2, CH), jnp.int32),
                       pltpu.VMEM((2, CH, H), jnp.float32),
                       pltpu.VMEM((N,), jnp.float32),
                       pltpu.VMEM((2, CH), jnp.float32),
                       pltpu.SemaphoreType.DMA,
                       pltpu.SemaphoreType.DMA],
    )(x1, ar, src, dst)


def _sc_scatter0_body(msg_hbm, ex_hbm, dst_hbm, acc_out, den_out,
                      rows_v, ex_v, didx_v, zrow_v, zden_v,
                      acc_sh, den_sh, lsem):
    c = lax.axis_index("c")
    s = lax.axis_index("s")
    _zero_shared(s, zrow_v, zden_v, acc_sh, den_sh)
    plsc.subcore_barrier()

    base = (c * NTL + s) * EPW

    def group_load(k, b):
        off = base + k * CH
        pltpu.async_copy(dst_hbm.at[pl.ds(off, CH)], didx_v.at[b], lsem)
        pltpu.async_copy(msg_hbm.at[pl.ds(off, CH)], rows_v.at[b], lsem)
        pltpu.async_copy(ex_hbm.at[pl.ds(off, CH)], ex_v.at[b], lsem)

    def group_wait(b):
        pltpu.make_async_copy(dst_hbm.at[pl.ds(0, CH)], didx_v.at[b],
                              lsem).wait()
        pltpu.make_async_copy(msg_hbm.at[pl.ds(0, CH)], rows_v.at[b],
                              lsem).wait()
        pltpu.make_async_copy(ex_hbm.at[pl.ds(0, CH)], ex_v.at[b],
                              lsem).wait()

    group_load(0, 0)
    group_load(1, 1)

    def body(g, carry):
        for b in range(2):
            k = g * 2 + b

            @pl.when(k < NCH)
            def _():
                group_wait(b)
                pltpu.sync_copy(rows_v.at[b], acc_sh.at[didx_v.at[b]],
                                add=True)
                pltpu.sync_copy(ex_v.at[b], den_sh.at[didx_v.at[b]],
                                add=True)

                @pl.when(k + 2 < NCH)
                def _():
                    group_load(k + 2, b)
        return carry

    lax.fori_loop(0, (NCH + 1) // 2, body, 0)
    plsc.subcore_barrier()
    pltpu.sync_copy(acc_sh.at[pl.ds(s * RPT, RPT)],
                    acc_out.at[c, pl.ds(s * RPT, RPT)])
    pltpu.sync_copy(den_sh.at[pl.ds(s * RPT, RPT)],
                    den_out.at[c, pl.ds(s * RPT, RPT)])


def _sc_scatter0(msgw, ex, dst):
    return pl.kernel(
        _sc_scatter0_body,
        out_type=[jax.ShapeDtypeStruct((NSC, NPAD, H), jnp.float32),
                  jax.ShapeDtypeStruct((NSC, NPAD), jnp.float32)],
        mesh=_sc_mesh(),
        compiler_params=pltpu.CompilerParams(needs_layout_passes=False),
        scratch_types=[pltpu.VMEM((2, CH, H), jnp.float32),
                       pltpu.VMEM((2, CH), jnp.float32),
                       pltpu.VMEM((2, CH), jnp.int32),
                       pltpu.VMEM((16, H), jnp.float32),
                       pltpu.VMEM((RPT,), jnp.float32),
                       pltpu.VMEM_SHARED((NPAD, H), jnp.float32),
                       pltpu.VMEM_SHARED((NPAD,), jnp.float32),
                       pltpu.SemaphoreType.DMA],
    )(msgw, ex, dst)


def _sc_gat_body(xl_hbm, as_hbm, ad_hbm, src_hbm, dst_hbm, acc_out, den_out,
                 sidx_v, didx_v, rows_v, ex_v, as_tab, ad_tab,
                 zrow_v, zden_v, acc_sh, den_sh, gsem, isem):
    c = lax.axis_index("c")
    s = lax.axis_index("s")
    _zero_shared(s, zrow_v, zden_v, acc_sh, den_sh)
    pltpu.sync_copy(as_hbm, as_tab)
    pltpu.sync_copy(ad_hbm, ad_tab)
    plsc.subcore_barrier()

    base = (c * NTL + s) * EPW

    def gather(b):
        pltpu.async_copy(xl_hbm.at[sidx_v.at[b]], rows_v.at[b], gsem)

    def gwait(b):
        pltpu.make_async_copy(xl_hbm.at[sidx_v.at[b]], rows_v.at[b],
                              gsem).wait()

    _idx_load_sync(src_hbm, dst_hbm, sidx_v, didx_v, base, 0)
    _idx_load_sync(src_hbm, dst_hbm, sidx_v, didx_v, base + CH, 1)
    gather(0)

    def body(g, carry):
        for b in range(2):
            k = g * 2 + b

            @pl.when(k < NCH)
            def _():
                bo = (b + 1) % 2
                gwait(b)

                @pl.when(k + 1 < NCH)
                def _():
                    @pl.when(k + 1 >= 2)
                    def _():
                        _idx_wait(src_hbm, dst_hbm, sidx_v, didx_v, bo, isem)
                    gather(bo)

                for j in range(CH // 16):
                    sid = sidx_v[b, pl.ds(j * 16, 16)]
                    did = didx_v[b, pl.ds(j * 16, 16)]
                    a = plsc.load_gather(as_tab, [sid]) \
                        + plsc.load_gather(ad_tab, [did])
                    a = jnp.where(a >= 0, a, 0.01 * a)
                    ex16 = jnp.exp(a)
                    ex_v[b, pl.ds(j * 16, 16)] = ex16
                    for t in range(16):
                        e = j * 16 + t
                        w = ex16[t]
                        for cc in range(H // 16):
                            rows_v[b, e, pl.ds(cc * 16, 16)] = \
                                rows_v[b, e, pl.ds(cc * 16, 16)] * w
                pltpu.sync_copy(rows_v.at[b], acc_sh.at[didx_v.at[b]],
                                add=True)
                pltpu.sync_copy(ex_v.at[b], den_sh.at[didx_v.at[b]],
                                add=True)

                @pl.when(k + 2 < NCH)
                def _():
                    _idx_load(src_hbm, dst_hbm, sidx_v, didx_v,
                              base + (k + 2) * CH, b, isem)
        return carry

    lax.fori_loop(0, (NCH + 1) // 2, body, 0)
    plsc.subcore_barrier()
    pltpu.sync_copy(acc_sh.at[pl.ds(s * RPT, RPT)],
                    acc_out.at[c, pl.ds(s * RPT, RPT)])
    pltpu.sync_copy(den_sh.at[pl.ds(s * RPT, RPT)],
                    den_out.at[c, pl.ds(s * RPT, RPT)])


def _sc_gat(xl, asrc, adst, src, dst):
    return pl.kernel(
        _sc_gat_body,
        out_type=[jax.ShapeDtypeStruct((NSC, NPAD, H), jnp.float32),
                  jax.ShapeDtypeStruct((NSC, NPAD), jnp.float32)],
        mesh=_sc_mesh(),
        compiler_params=pltpu.CompilerParams(needs_layout_passes=False),
        scratch_types=[pltpu.VMEM((2, CH), jnp.int32),
                       pltpu.VMEM((2, CH), jnp.int32),
                       pltpu.VMEM((2, CH, H), jnp.float32),
                       pltpu.VMEM((2, CH), jnp.float32),
                       pltpu.VMEM((N,), jnp.float32),
                       pltpu.VMEM((N,), jnp.float32),
                       pltpu.VMEM((16, H), jnp.float32),
                       pltpu.VMEM((RPT,), jnp.float32),
                       pltpu.VMEM_SHARED((NPAD, H), jnp.float32),
                       pltpu.VMEM_SHARED((NPAD,), jnp.float32),
                       pltpu.SemaphoreType.DMA,
                       pltpu.SemaphoreType.DMA],
    )(xl, asrc, adst, src, dst)


def kernel(x, edge_index, edge_attr, params):
    p = params
    src = edge_index[0]
    dst = edge_index[1]

    x1, ar, xw = _node_enter(x, p["lin_enter_W"].T, p["lin_enter_b"][None],
                             p["att_r"], p["lin1_W"][:, :H].T)

    xjw, ardst = _sc_gather0(xw, ar.reshape(N), src, dst)
    msgw, ex = _edge0(xjw, edge_attr, ardst.reshape(E, 1),
                      p["lin1_W"][:, H:].T, p["att_l"])
    acc, den = _sc_scatter0(msgw, ex.reshape(E), dst)
    xn, xl, asrc, adst = _combine_mid(
        NSC, acc, den.reshape(NSC, NPAD, 1), p["conv_bias"][None], x1,
        p["gru0_w_ih"].T, p["gru0_w_hh"].T,
        p["gru0_b_ih"][None], p["gru0_b_hh"][None],
        p["gat1_W"].T, p["gat1_att_src"], p["gat1_att_dst"],
        preW=p["lin2_W"].T)

    for l in (1, 2):
        accl, denl = _sc_gat(xl, asrc.reshape(N), adst.reshape(N), src, dst)
        denl = denl.reshape(NSC, NPAD, 1)
        if l == 1:
            xn, xl, asrc, adst = _combine_mid(
                NSC, accl, denl, p["gat1_bias"][None], xn,
                p["gru1_w_ih"].T, p["gru1_w_hh"].T,
                p["gru1_b_ih"][None], p["gru1_b_hh"][None],
                p["gat2_W"].T, p["gat2_att_src"], p["gat2_att_dst"])
        else:
            y = _combine_final(
                NSC, accl, denl, p["gat2_bias"][None], xn,
                p["gru2_w_ih"].T, p["gru2_w_hh"].T,
                p["gru2_b_ih"][None], p["gru2_b_hh"][None])
    return y
